# baseline (speedup 1.0000x reference)
import functools

import jax
import jax.numpy as jnp
from jax import lax
from jax.experimental import pallas as pl
from jax.experimental.pallas import tpu as pltpu
from jax.experimental.pallas import tpu_sc as plsc

N = 10000
E = 320000
D = 128
H = 128
G = 64
EPS = 1e-5

NC = 2
NS = 16
NW = NC * NS
CH = 128
NCH = 164
NBUF = 1
HF = H // 2
EPW = NCH * CH
TOT = NS * EPW
N_PAD = 10240
RPT = N_PAD // NS
N_PAD_A = 10112
RPT_A = N_PAD_A // NS
NB = 25
RB = 400

_mesh = plsc.VectorSubcoreMesh(core_axis_name="c", subcore_axis_name="s")


@functools.partial(
    pl.kernel,
    out_type=jax.ShapeDtypeStruct((NC, N_PAD), jnp.float32),
    mesh=_mesh,
    scratch_types=[
        pltpu.VMEM_SHARED((N_PAD,), jnp.float32),
        pltpu.VMEM((NCH // 2, CH), jnp.int32),
        pltpu.VMEM((CH,), jnp.float32),
    ],
)
def _deg_kernel(dst_hbm, zcol_hbm, out_hbm, deg_sh, dstv, onesv):
    c = lax.axis_index("c")
    s = lax.axis_index("s")
    wid = c * NS + s
    pltpu.sync_copy(zcol_hbm, deg_sh.at[pl.ds(s * RPT, RPT)])
    pltpu.sync_copy(dst_hbm.at[wid], dstv)
    for i in range(CH // 16):
        onesv[pl.ds(i * 16, 16)] = jnp.ones((16,), jnp.float32)
    plsc.subcore_barrier()

    def body(j, carry):
        pltpu.sync_copy(onesv, deg_sh.at[dstv.at[j]], add=True)
        return carry

    lax.fori_loop(0, NCH // 2, body, 0)
    plsc.subcore_barrier()
    pltpu.sync_copy(deg_sh.at[pl.ds(s * RPT, RPT)],
                    out_hbm.at[c, pl.ds(s * RPT, RPT)])


@functools.partial(
    pl.kernel,
    out_type=jax.ShapeDtypeStruct((NC, N_PAD_A, HF), jnp.float32),
    mesh=_mesh,
    compiler_params=pltpu.CompilerParams(use_tc_tiling_on_sc=False),
    scratch_types=[
        pltpu.VMEM_SHARED((N_PAD_A, HF), jnp.float32),
        pltpu.VMEM((NCH, CH), jnp.int32),
        pltpu.VMEM((NCH, CH), jnp.int32),
        [pltpu.VMEM((CH, HF), jnp.float32) for _ in range(NBUF)],
        [pltpu.SemaphoreType.DMA for _ in range(NBUF)],
    ],
)
def _scatter_kernel(hp_hbm, src_hbm, dst_hbm, zrows_hbm, out_hbm,
                    acc_sh, srcv, dstv, rows, gsems):
    c = lax.axis_index("c")
    s = lax.axis_index("s")
    pltpu.sync_copy(src_hbm.at[c, s], srcv)
    pltpu.sync_copy(dst_hbm.at[s], dstv)
    pltpu.sync_copy(zrows_hbm, acc_sh.at[pl.ds(s * RPT_A, RPT_A)])
    for b in range(NBUF):
        pltpu.async_copy(hp_hbm.at[srcv.at[b]], rows[b], gsems[b])
    plsc.subcore_barrier()

    def body(i, carry):
        for b in range(NBUF):
            j = i * NBUF + b
            pltpu.make_async_copy(hp_hbm.at[srcv.at[j]], rows[b],
                                  gsems[b]).wait()
            pltpu.sync_copy(rows[b], acc_sh.at[dstv.at[j]], add=True)
            jn = j + NBUF

            @pl.when(jn < NCH)
            def _():
                pltpu.async_copy(hp_hbm.at[srcv.at[jn]], rows[b], gsems[b])

        return carry

    lax.fori_loop(0, NCH // NBUF, body, 0)
    plsc.subcore_barrier()
    pltpu.sync_copy(acc_sh.at[pl.ds(s * RPT_A, RPT_A)],
                    out_hbm.at[c, pl.ds(s * RPT_A, RPT_A)])


def _mm_body(x_ref, w_ref, deg_ref, o_ref):
    dinv = lax.rsqrt(deg_ref[0] + deg_ref[1])
    t = jnp.dot(x_ref[...], w_ref[...],
                preferred_element_type=jnp.float32) * dinv
    o_ref[0] = t[:, :HF]
    o_ref[1] = t[:, HF:]


def _mm_call(x, w, degs):
    return pl.pallas_call(
        _mm_body,
        grid=(NB,),
        in_specs=[
            pl.BlockSpec((RB, D), lambda i: (i, 0)),
            pl.BlockSpec((D, H), lambda i: (0, 0)),
            pl.BlockSpec((NC, RB, 1), lambda i: (0, i, 0)),
        ],
        out_specs=pl.BlockSpec((NC, RB, HF), lambda i: (0, i, 0)),
        out_shape=jax.ShapeDtypeStruct((NC, N, HF), jnp.float32),
    )(x, w, degs)


def _post_body(acc_ref, deg_ref, b_ref, conv_ref, st_ref, sums):
    i = pl.program_id(0)
    dinv = lax.rsqrt(deg_ref[0] + deg_ref[1])
    conv = jnp.concatenate([acc_ref[0], acc_ref[1]], axis=1) * dinv \
        + b_ref[...]
    conv_ref[...] = conv

    @pl.when(i == 0)
    def _():
        sums[...] = jnp.zeros_like(sums)

    sums[0:1, :] += jnp.sum(conv, axis=0, keepdims=True)
    sums[1:2, :] += jnp.sum(conv * conv, axis=0, keepdims=True)

    @pl.when(i == NB - 1)
    def _():
        m = sums[0:1, :] / N
        v = sums[1:2, :] / N - m * m
        st_ref[...] = jnp.concatenate([m, v], axis=0)


def _post_call(acc, degs, b):
    return pl.pallas_call(
        _post_body,
        grid=(NB,),
        in_specs=[
            pl.BlockSpec((NC, RB, HF), lambda i: (0, i, 0)),
            pl.BlockSpec((NC, RB, 1), lambda i: (0, i, 0)),
            pl.BlockSpec((1, H), lambda i: (0, 0)),
        ],
        out_specs=[
            pl.BlockSpec((RB, H), lambda i: (i, 0)),
            pl.BlockSpec((2, H), lambda i: (0, 0)),
        ],
        out_shape=[
            jax.ShapeDtypeStruct((N, H), jnp.float32),
            jax.ShapeDtypeStruct((2, H), jnp.float32),
        ],
        scratch_shapes=[pltpu.VMEM((2, H), jnp.float32)],
    )(acc, degs, b)


def _bnmm_body(conv_ref, st_ref, g_ref, be_ref, w_ref, deg_ref, o_ref):
    m = st_ref[0:1, :]
    v = st_ref[1:2, :]
    y = jnp.maximum((conv_ref[...] - m) * lax.rsqrt(v + EPS)
                    * g_ref[...] + be_ref[...], 0.0)
    dinv = lax.rsqrt(deg_ref[0] + deg_ref[1])
    t = jnp.dot(y, w_ref[...], preferred_element_type=jnp.float32) * dinv
    o_ref[0] = t[:, :HF]
    o_ref[1] = t[:, HF:]


def _bnmm_call(conv, st, g, be, w, degs):
    return pl.pallas_call(
        _bnmm_body,
        grid=(NB,),
        in_specs=[
            pl.BlockSpec((RB, H), lambda i: (i, 0)),
            pl.BlockSpec((2, H), lambda i: (0, 0)),
            pl.BlockSpec((1, H), lambda i: (0, 0)),
            pl.BlockSpec((1, H), lambda i: (0, 0)),
            pl.BlockSpec((H, H), lambda i: (0, 0)),
            pl.BlockSpec((NC, RB, 1), lambda i: (0, i, 0)),
        ],
        out_specs=pl.BlockSpec((NC, RB, HF), lambda i: (0, i, 0)),
        out_shape=jax.ShapeDtypeStruct((NC, N, HF), jnp.float32),
    )(conv, st, g, be, w, degs)


def _head_body(conv_ref, st_ref, g_ref, be_ref, batch_ref,
               lw1_ref, lb1_ref, lw2_ref, lb2_ref, o_ref, sums, cnt):
    i = pl.program_id(0)
    m = st_ref[0:1, :]
    v = st_ref[1:2, :]
    y = jnp.maximum((conv_ref[...] - m) * lax.rsqrt(v + EPS)
                    * g_ref[...] + be_ref[...], 0.0)

    @pl.when(i == 0)
    def _():
        sums[...] = jnp.zeros_like(sums)
        cnt[...] = jnp.zeros_like(cnt)

    onehot = (batch_ref[...] ==
              lax.broadcasted_iota(jnp.int32, (1, G), 1)).astype(jnp.float32)
    sums[...] += lax.dot_general(onehot, y, (((0,), (0,)), ((), ())),
                                 preferred_element_type=jnp.float32)
    cnt[...] += lax.dot_general(onehot, jnp.ones((RB, 1), jnp.float32),
                                (((0,), (0,)), ((), ())),
                                preferred_element_type=jnp.float32)

    @pl.when(i == NB - 1)
    def _():
        pooled = sums[...] / jnp.maximum(cnt[...], 1.0)
        h = jnp.maximum(jnp.dot(pooled, lw1_ref[...],
                                preferred_element_type=jnp.float32)
                        + lb1_ref[...], 0.0)
        o_ref[...] = jnp.dot(h, lw2_ref[...],
                             preferred_element_type=jnp.float32) + lb2_ref[...]


def _head_call(conv, st, g, be, batchcol, lw1, lb1, lw2, lb2):
    return pl.pallas_call(
        _head_body,
        grid=(NB,),
        in_specs=[
            pl.BlockSpec((RB, H), lambda i: (i, 0)),
            pl.BlockSpec((2, H), lambda i: (0, 0)),
            pl.BlockSpec((1, H), lambda i: (0, 0)),
            pl.BlockSpec((1, H), lambda i: (0, 0)),
            pl.BlockSpec((RB, 1), lambda i: (i, 0)),
            pl.BlockSpec((H, G), lambda i: (0, 0)),
            pl.BlockSpec((1, G), lambda i: (0, 0)),
            pl.BlockSpec((G, 1), lambda i: (0, 0)),
            pl.BlockSpec((1, 1), lambda i: (0, 0)),
        ],
        out_specs=pl.BlockSpec((G, 1), lambda i: (0, 0)),
        out_shape=jax.ShapeDtypeStruct((G, 1), jnp.float32),
        scratch_shapes=[
            pltpu.VMEM((G, H), jnp.float32),
            pltpu.VMEM((G, 1), jnp.float32),
        ],
    )(conv, st, g, be, batchcol, lw1, lb1, lw2, lb2)


def kernel(x, edge_index, batch, W1, b1, W2, b2, W3, b3,
           g1, be1, g2, be2, g3, be3, lw1, lb1, lw2, lb2):
    loop = jnp.arange(N, dtype=jnp.int32)
    pad = TOT - (E + N)
    src_t = jnp.concatenate(
        [edge_index[0], loop, jnp.zeros((pad,), jnp.int32)]).reshape(NS, NCH, CH)
    src_all = jnp.stack([src_t, src_t + N])
    dst_all = jnp.concatenate(
        [edge_index[1], loop, jnp.full((pad,), N, jnp.int32)]).reshape(NS, NCH, CH)
    zcol = jnp.zeros((RPT,), jnp.float32)
    zrows = jnp.zeros((RPT_A, HF), jnp.float32)

    degs = _deg_kernel(dst_all.reshape(NW, NCH // 2, CH),
                       zcol).reshape(NC, N_PAD, 1)

    hp = _mm_call(x, W1, degs)
    acc = _scatter_kernel(hp.reshape(NC * N, HF), src_all, dst_all, zrows)
    conv, st = _post_call(acc, degs, b1.reshape(1, H))

    hp = _bnmm_call(conv, st, g1.reshape(1, H), be1.reshape(1, H), W2, degs)
    acc = _scatter_kernel(hp.reshape(NC * N, HF), src_all, dst_all, zrows)
    conv, st = _post_call(acc, degs, b2.reshape(1, H))

    hp = _bnmm_call(conv, st, g2.reshape(1, H), be2.reshape(1, H), W3, degs)
    acc = _scatter_kernel(hp.reshape(NC * N, HF), src_all, dst_all, zrows)
    conv, st = _post_call(acc, degs, b3.reshape(1, H))

    return _head_call(conv, st, g3.reshape(1, H), be3.reshape(1, H),
                      batch.reshape(N, 1).astype(jnp.int32),
                      lw1, lb1.reshape(1, G), lw2, lb2.reshape(1, 1))


# speedup vs baseline: 1.3165x; 1.3165x over previous
import functools

import jax
import jax.numpy as jnp
from jax import lax
from jax.experimental import pallas as pl
from jax.experimental.pallas import tpu as pltpu
from jax.experimental.pallas import tpu_sc as plsc

N = 10000
E = 320000
D = 128
H = 128
G = 64
EPS = 1e-5

NC = 2
NS = 16
NW = NC * NS
CH = 128
NCH = 164
NBUF = 4
HF = H // 2
EPW = NCH * CH
TOT = NS * EPW
N_PAD = 10240
RPT = N_PAD // NS
N_PAD_A = 10112
RPT_A = N_PAD_A // NS
NB = 25
RB = 400

_mesh = plsc.VectorSubcoreMesh(core_axis_name="c", subcore_axis_name="s")


@functools.partial(
    pl.kernel,
    out_type=jax.ShapeDtypeStruct((NC, N_PAD), jnp.float32),
    mesh=_mesh,
    scratch_types=[
        pltpu.VMEM_SHARED((N_PAD,), jnp.float32),
        pltpu.VMEM((NCH // 2, CH), jnp.int32),
        pltpu.VMEM((CH,), jnp.float32),
    ],
)
def _deg_kernel(dst_hbm, zcol_hbm, out_hbm, deg_sh, dstv, onesv):
    c = lax.axis_index("c")
    s = lax.axis_index("s")
    wid = c * NS + s
    pltpu.sync_copy(zcol_hbm, deg_sh.at[pl.ds(s * RPT, RPT)])
    pltpu.sync_copy(dst_hbm.at[wid], dstv)
    for i in range(CH // 16):
        onesv[pl.ds(i * 16, 16)] = jnp.ones((16,), jnp.float32)
    plsc.subcore_barrier()

    def body(j, carry):
        pltpu.sync_copy(onesv, deg_sh.at[dstv.at[j]], add=True)
        return carry

    lax.fori_loop(0, NCH // 2, body, 0)
    plsc.subcore_barrier()
    pltpu.sync_copy(deg_sh.at[pl.ds(s * RPT, RPT)],
                    out_hbm.at[c, pl.ds(s * RPT, RPT)])


@functools.partial(
    pl.kernel,
    out_type=jax.ShapeDtypeStruct((NC, N_PAD_A, HF), jnp.float32),
    mesh=_mesh,
    compiler_params=pltpu.CompilerParams(use_tc_tiling_on_sc=False),
    scratch_types=[
        pltpu.VMEM_SHARED((N_PAD_A, HF), jnp.float32),
        pltpu.VMEM((NCH, CH), jnp.int32),
        pltpu.VMEM((NCH, CH), jnp.int32),
        [pltpu.VMEM((CH, HF), jnp.float32) for _ in range(NBUF)],
        [pltpu.SemaphoreType.DMA for _ in range(NBUF)],
        [pltpu.SemaphoreType.DMA for _ in range(NBUF)],
    ],
)
def _scatter_kernel(hp_hbm, src_hbm, dst_hbm, zrows_hbm, out_hbm,
                    acc_sh, srcv, dstv, rows, gsems, ssems):
    c = lax.axis_index("c")
    s = lax.axis_index("s")
    pltpu.sync_copy(src_hbm.at[c, s], srcv)
    pltpu.sync_copy(dst_hbm.at[s], dstv)
    pltpu.sync_copy(zrows_hbm, acc_sh.at[pl.ds(s * RPT_A, RPT_A)])
    for b in range(NBUF):
        pltpu.async_copy(hp_hbm.at[srcv.at[b]], rows[b], gsems[b])
    plsc.subcore_barrier()

    def body(i, carry):
        for b in range(NBUF):
            j = i * NBUF + b
            bp = (b - 1) % NBUF
            pltpu.make_async_copy(hp_hbm.at[srcv.at[j]], rows[b],
                                  gsems[b]).wait()
            pltpu.async_copy(rows[b], acc_sh.at[dstv.at[j]], ssems[b],
                             add=True)
            jp = j - 1 + NBUF

            @pl.when((j >= 1) & (jp < NCH))
            def _():
                pltpu.make_async_copy(rows[bp], acc_sh.at[dstv.at[0]],
                                      ssems[bp]).wait()
                pltpu.async_copy(hp_hbm.at[srcv.at[jp]], rows[bp], gsems[bp])

        return carry

    lax.fori_loop(0, NCH // NBUF, body, 0)
    for b in range(NBUF):
        pltpu.make_async_copy(rows[b], acc_sh.at[dstv.at[0]],
                              ssems[b]).wait()
    plsc.subcore_barrier()
    pltpu.sync_copy(acc_sh.at[pl.ds(s * RPT_A, RPT_A)],
                    out_hbm.at[c, pl.ds(s * RPT_A, RPT_A)])


def _mm_body(x_ref, w_ref, deg_ref, o_ref):
    dinv = lax.rsqrt(deg_ref[0] + deg_ref[1])
    t = jnp.dot(x_ref[...], w_ref[...],
                preferred_element_type=jnp.float32) * dinv
    o_ref[0] = t[:, :HF]
    o_ref[1] = t[:, HF:]


def _mm_call(x, w, degs):
    return pl.pallas_call(
        _mm_body,
        grid=(NB,),
        in_specs=[
            pl.BlockSpec((RB, D), lambda i: (i, 0)),
            pl.BlockSpec((D, H), lambda i: (0, 0)),
            pl.BlockSpec((NC, RB, 1), lambda i: (0, i, 0)),
        ],
        out_specs=pl.BlockSpec((NC, RB, HF), lambda i: (0, i, 0)),
        out_shape=jax.ShapeDtypeStruct((NC, N, HF), jnp.float32),
    )(x, w, degs)


def _post_body(acc_ref, deg_ref, b_ref, conv_ref, st_ref, sums):
    i = pl.program_id(0)
    dinv = lax.rsqrt(deg_ref[0] + deg_ref[1])
    conv = jnp.concatenate([acc_ref[0], acc_ref[1]], axis=1) * dinv \
        + b_ref[...]
    conv_ref[...] = conv

    @pl.when(i == 0)
    def _():
        sums[...] = jnp.zeros_like(sums)

    sums[0:1, :] += jnp.sum(conv, axis=0, keepdims=True)
    sums[1:2, :] += jnp.sum(conv * conv, axis=0, keepdims=True)

    @pl.when(i == NB - 1)
    def _():
        m = sums[0:1, :] / N
        v = sums[1:2, :] / N - m * m
        st_ref[...] = jnp.concatenate([m, v], axis=0)


def _post_call(acc, degs, b):
    return pl.pallas_call(
        _post_body,
        grid=(NB,),
        in_specs=[
            pl.BlockSpec((NC, RB, HF), lambda i: (0, i, 0)),
            pl.BlockSpec((NC, RB, 1), lambda i: (0, i, 0)),
            pl.BlockSpec((1, H), lambda i: (0, 0)),
        ],
        out_specs=[
            pl.BlockSpec((RB, H), lambda i: (i, 0)),
            pl.BlockSpec((2, H), lambda i: (0, 0)),
        ],
        out_shape=[
            jax.ShapeDtypeStruct((N, H), jnp.float32),
            jax.ShapeDtypeStruct((2, H), jnp.float32),
        ],
        scratch_shapes=[pltpu.VMEM((2, H), jnp.float32)],
    )(acc, degs, b)


def _bnmm_body(conv_ref, st_ref, g_ref, be_ref, w_ref, deg_ref, o_ref):
    m = st_ref[0:1, :]
    v = st_ref[1:2, :]
    y = jnp.maximum((conv_ref[...] - m) * lax.rsqrt(v + EPS)
                    * g_ref[...] + be_ref[...], 0.0)
    dinv = lax.rsqrt(deg_ref[0] + deg_ref[1])
    t = jnp.dot(y, w_ref[...], preferred_element_type=jnp.float32) * dinv
    o_ref[0] = t[:, :HF]
    o_ref[1] = t[:, HF:]


def _bnmm_call(conv, st, g, be, w, degs):
    return pl.pallas_call(
        _bnmm_body,
        grid=(NB,),
        in_specs=[
            pl.BlockSpec((RB, H), lambda i: (i, 0)),
            pl.BlockSpec((2, H), lambda i: (0, 0)),
            pl.BlockSpec((1, H), lambda i: (0, 0)),
            pl.BlockSpec((1, H), lambda i: (0, 0)),
            pl.BlockSpec((H, H), lambda i: (0, 0)),
            pl.BlockSpec((NC, RB, 1), lambda i: (0, i, 0)),
        ],
        out_specs=pl.BlockSpec((NC, RB, HF), lambda i: (0, i, 0)),
        out_shape=jax.ShapeDtypeStruct((NC, N, HF), jnp.float32),
    )(conv, st, g, be, w, degs)


def _head_body(conv_ref, st_ref, g_ref, be_ref, batch_ref,
               lw1_ref, lb1_ref, lw2_ref, lb2_ref, o_ref, sums, cnt):
    i = pl.program_id(0)
    m = st_ref[0:1, :]
    v = st_ref[1:2, :]
    y = jnp.maximum((conv_ref[...] - m) * lax.rsqrt(v + EPS)
                    * g_ref[...] + be_ref[...], 0.0)

    @pl.when(i == 0)
    def _():
        sums[...] = jnp.zeros_like(sums)
        cnt[...] = jnp.zeros_like(cnt)

    onehot = (batch_ref[...] ==
              lax.broadcasted_iota(jnp.int32, (1, G), 1)).astype(jnp.float32)
    sums[...] += lax.dot_general(onehot, y, (((0,), (0,)), ((), ())),
                                 preferred_element_type=jnp.float32)
    cnt[...] += lax.dot_general(onehot, jnp.ones((RB, 1), jnp.float32),
                                (((0,), (0,)), ((), ())),
                                preferred_element_type=jnp.float32)

    @pl.when(i == NB - 1)
    def _():
        pooled = sums[...] / jnp.maximum(cnt[...], 1.0)
        h = jnp.maximum(jnp.dot(pooled, lw1_ref[...],
                                preferred_element_type=jnp.float32)
                        + lb1_ref[...], 0.0)
        o_ref[...] = jnp.dot(h, lw2_ref[...],
                             preferred_element_type=jnp.float32) + lb2_ref[...]


def _head_call(conv, st, g, be, batchcol, lw1, lb1, lw2, lb2):
    return pl.pallas_call(
        _head_body,
        grid=(NB,),
        in_specs=[
            pl.BlockSpec((RB, H), lambda i: (i, 0)),
            pl.BlockSpec((2, H), lambda i: (0, 0)),
            pl.BlockSpec((1, H), lambda i: (0, 0)),
            pl.BlockSpec((1, H), lambda i: (0, 0)),
            pl.BlockSpec((RB, 1), lambda i: (i, 0)),
            pl.BlockSpec((H, G), lambda i: (0, 0)),
            pl.BlockSpec((1, G), lambda i: (0, 0)),
            pl.BlockSpec((G, 1), lambda i: (0, 0)),
            pl.BlockSpec((1, 1), lambda i: (0, 0)),
        ],
        out_specs=pl.BlockSpec((G, 1), lambda i: (0, 0)),
        out_shape=jax.ShapeDtypeStruct((G, 1), jnp.float32),
        scratch_shapes=[
            pltpu.VMEM((G, H), jnp.float32),
            pltpu.VMEM((G, 1), jnp.float32),
        ],
    )(conv, st, g, be, batchcol, lw1, lb1, lw2, lb2)


def kernel(x, edge_index, batch, W1, b1, W2, b2, W3, b3,
           g1, be1, g2, be2, g3, be3, lw1, lb1, lw2, lb2):
    loop = jnp.arange(N, dtype=jnp.int32)
    pad = TOT - (E + N)
    src_t = jnp.concatenate(
        [edge_index[0], loop, jnp.zeros((pad,), jnp.int32)]).reshape(NS, NCH, CH)
    src_all = jnp.stack([src_t, src_t + N])
    dst_all = jnp.concatenate(
        [edge_index[1], loop, jnp.full((pad,), N, jnp.int32)]).reshape(NS, NCH, CH)
    zcol = jnp.zeros((RPT,), jnp.float32)
    zrows = jnp.zeros((RPT_A, HF), jnp.float32)

    degs = _deg_kernel(dst_all.reshape(NW, NCH // 2, CH),
                       zcol).reshape(NC, N_PAD, 1)

    hp = _mm_call(x, W1, degs)
    acc = _scatter_kernel(hp.reshape(NC * N, HF), src_all, dst_all, zrows)
    conv, st = _post_call(acc, degs, b1.reshape(1, H))

    hp = _bnmm_call(conv, st, g1.reshape(1, H), be1.reshape(1, H), W2, degs)
    acc = _scatter_kernel(hp.reshape(NC * N, HF), src_all, dst_all, zrows)
    conv, st = _post_call(acc, degs, b2.reshape(1, H))

    hp = _bnmm_call(conv, st, g2.reshape(1, H), be2.reshape(1, H), W3, degs)
    acc = _scatter_kernel(hp.reshape(NC * N, HF), src_all, dst_all, zrows)
    conv, st = _post_call(acc, degs, b3.reshape(1, H))

    return _head_call(conv, st, g3.reshape(1, H), be3.reshape(1, H),
                      batch.reshape(N, 1).astype(jnp.int32),
                      lw1, lb1.reshape(1, G), lw2, lb2.reshape(1, 1))


# speedup vs baseline: 2.2791x; 1.7312x over previous
import functools

import jax
import jax.numpy as jnp
from jax import lax
from jax.experimental import pallas as pl
from jax.experimental.pallas import tpu as pltpu
from jax.experimental.pallas import tpu_sc as plsc

N = 10000
E = 320000
D = 128
H = 128
G = 64
EPS = 1e-5

NC = 2
NS = 16
NW = NC * NS
CH = 128
NCH = 164
NBUF = 4
HF = H // 2
EPW = NCH * CH
TOT = NS * EPW
N_PAD = 10240
RPT = N_PAD // NS
N_PAD_A = 10112
RPT_A = N_PAD_A // NS
NB = 25
RB = 400

_mesh = plsc.VectorSubcoreMesh(core_axis_name="c", subcore_axis_name="s")


@functools.partial(
    pl.kernel,
    out_type=jax.ShapeDtypeStruct((NC, N_PAD), jnp.float32),
    mesh=_mesh,
    scratch_types=[
        pltpu.VMEM_SHARED((N_PAD,), jnp.float32),
        pltpu.VMEM((NCH // 2, CH), jnp.int32),
        pltpu.VMEM((CH,), jnp.float32),
    ],
)
def _deg_kernel(dst_hbm, zcol_hbm, out_hbm, deg_sh, dstv, onesv):
    c = lax.axis_index("c")
    s = lax.axis_index("s")
    wid = c * NS + s
    pltpu.sync_copy(zcol_hbm, deg_sh.at[pl.ds(s * RPT, RPT)])
    pltpu.sync_copy(dst_hbm.at[wid], dstv)
    for i in range(CH // 16):
        onesv[pl.ds(i * 16, 16)] = jnp.ones((16,), jnp.float32)
    plsc.subcore_barrier()

    def body(j, carry):
        pltpu.sync_copy(onesv, deg_sh.at[dstv.at[j]], add=True)
        return carry

    lax.fori_loop(0, NCH // 2, body, 0)
    plsc.subcore_barrier()
    pltpu.sync_copy(deg_sh.at[pl.ds(s * RPT, RPT)],
                    out_hbm.at[c, pl.ds(s * RPT, RPT)])


@functools.partial(
    pl.kernel,
    out_type=jax.ShapeDtypeStruct((NC, N_PAD_A, HF), jnp.bfloat16),
    mesh=_mesh,
    compiler_params=pltpu.CompilerParams(use_tc_tiling_on_sc=False),
    scratch_types=[
        pltpu.VMEM_SHARED((N_PAD_A, HF), jnp.bfloat16),
        pltpu.VMEM((NCH, CH), jnp.int32),
        pltpu.VMEM((NCH, CH), jnp.int32),
        [pltpu.VMEM((CH, HF), jnp.bfloat16) for _ in range(NBUF)],
        [pltpu.SemaphoreType.DMA for _ in range(NBUF)],
        [pltpu.SemaphoreType.DMA for _ in range(NBUF)],
    ],
)
def _scatter_kernel(hp_hbm, src_hbm, dst_hbm, zrows_hbm, out_hbm,
                    acc_sh, srcv, dstv, rows, gsems, ssems):
    c = lax.axis_index("c")
    s = lax.axis_index("s")
    pltpu.sync_copy(src_hbm.at[c, s], srcv)
    pltpu.sync_copy(dst_hbm.at[s], dstv)
    pltpu.sync_copy(zrows_hbm, acc_sh.at[pl.ds(s * RPT_A, RPT_A)])
    for b in range(NBUF):
        pltpu.async_copy(hp_hbm.at[srcv.at[b]], rows[b], gsems[b])
    plsc.subcore_barrier()

    def body(i, carry):
        for b in range(NBUF):
            j = i * NBUF + b
            bp = (b - 1) % NBUF
            pltpu.make_async_copy(hp_hbm.at[srcv.at[j]], rows[b],
                                  gsems[b]).wait()
            pltpu.async_copy(rows[b], acc_sh.at[dstv.at[j]], ssems[b],
                             add=True)
            jp = j - 1 + NBUF

            @pl.when((j >= 1) & (jp < NCH))
            def _():
                pltpu.make_async_copy(rows[bp], acc_sh.at[dstv.at[0]],
                                      ssems[bp]).wait()
                pltpu.async_copy(hp_hbm.at[srcv.at[jp]], rows[bp], gsems[bp])

        return carry

    lax.fori_loop(0, NCH // NBUF, body, 0)
    for b in range(NBUF):
        pltpu.make_async_copy(rows[b], acc_sh.at[dstv.at[0]],
                              ssems[b]).wait()
    plsc.subcore_barrier()
    pltpu.sync_copy(acc_sh.at[pl.ds(s * RPT_A, RPT_A)],
                    out_hbm.at[c, pl.ds(s * RPT_A, RPT_A)])


def _mm_body(x_ref, w_ref, deg_ref, o_ref):
    dinv = lax.rsqrt(deg_ref[0] + deg_ref[1])
    t = (jnp.dot(x_ref[...], w_ref[...],
                 preferred_element_type=jnp.float32)
         * dinv).astype(jnp.bfloat16)
    o_ref[0] = t[:, :HF]
    o_ref[1] = t[:, HF:]


def _mm_call(x, w, degs):
    return pl.pallas_call(
        _mm_body,
        grid=(NB,),
        in_specs=[
            pl.BlockSpec((RB, D), lambda i: (i, 0)),
            pl.BlockSpec((D, H), lambda i: (0, 0)),
            pl.BlockSpec((NC, RB, 1), lambda i: (0, i, 0)),
        ],
        out_specs=pl.BlockSpec((NC, RB, HF), lambda i: (0, i, 0)),
        out_shape=jax.ShapeDtypeStruct((NC, N, HF), jnp.bfloat16),
    )(x, w, degs)


def _post_body(acc_ref, deg_ref, b_ref, conv_ref, st_ref, sums):
    i = pl.program_id(0)
    dinv = lax.rsqrt(deg_ref[0] + deg_ref[1])
    conv = jnp.concatenate([acc_ref[0], acc_ref[1]],
                           axis=1).astype(jnp.float32) * dinv + b_ref[...]
    conv_ref[...] = conv

    @pl.when(i == 0)
    def _():
        sums[...] = jnp.zeros_like(sums)

    sums[0:1, :] += jnp.sum(conv, axis=0, keepdims=True)
    sums[1:2, :] += jnp.sum(conv * conv, axis=0, keepdims=True)

    @pl.when(i == NB - 1)
    def _():
        m = sums[0:1, :] / N
        v = sums[1:2, :] / N - m * m
        st_ref[...] = jnp.concatenate([m, v], axis=0)


def _post_call(acc, degs, b):
    return pl.pallas_call(
        _post_body,
        grid=(NB,),
        in_specs=[
            pl.BlockSpec((NC, RB, HF), lambda i: (0, i, 0)),
            pl.BlockSpec((NC, RB, 1), lambda i: (0, i, 0)),
            pl.BlockSpec((1, H), lambda i: (0, 0)),
        ],
        out_specs=[
            pl.BlockSpec((RB, H), lambda i: (i, 0)),
            pl.BlockSpec((2, H), lambda i: (0, 0)),
        ],
        out_shape=[
            jax.ShapeDtypeStruct((N, H), jnp.float32),
            jax.ShapeDtypeStruct((2, H), jnp.float32),
        ],
        scratch_shapes=[pltpu.VMEM((2, H), jnp.float32)],
    )(acc, degs, b)


def _bnmm_body(conv_ref, st_ref, g_ref, be_ref, w_ref, deg_ref, o_ref):
    m = st_ref[0:1, :]
    v = st_ref[1:2, :]
    y = jnp.maximum((conv_ref[...] - m) * lax.rsqrt(v + EPS)
                    * g_ref[...] + be_ref[...], 0.0)
    dinv = lax.rsqrt(deg_ref[0] + deg_ref[1])
    t = (jnp.dot(y, w_ref[...], preferred_element_type=jnp.float32)
         * dinv).astype(jnp.bfloat16)
    o_ref[0] = t[:, :HF]
    o_ref[1] = t[:, HF:]


def _bnmm_call(conv, st, g, be, w, degs):
    return pl.pallas_call(
        _bnmm_body,
        grid=(NB,),
        in_specs=[
            pl.BlockSpec((RB, H), lambda i: (i, 0)),
            pl.BlockSpec((2, H), lambda i: (0, 0)),
            pl.BlockSpec((1, H), lambda i: (0, 0)),
            pl.BlockSpec((1, H), lambda i: (0, 0)),
            pl.BlockSpec((H, H), lambda i: (0, 0)),
            pl.BlockSpec((NC, RB, 1), lambda i: (0, i, 0)),
        ],
        out_specs=pl.BlockSpec((NC, RB, HF), lambda i: (0, i, 0)),
        out_shape=jax.ShapeDtypeStruct((NC, N, HF), jnp.bfloat16),
    )(conv, st, g, be, w, degs)


def _head_body(conv_ref, st_ref, g_ref, be_ref, batch_ref,
               lw1_ref, lb1_ref, lw2_ref, lb2_ref, o_ref, sums, cnt):
    i = pl.program_id(0)
    m = st_ref[0:1, :]
    v = st_ref[1:2, :]
    y = jnp.maximum((conv_ref[...] - m) * lax.rsqrt(v + EPS)
                    * g_ref[...] + be_ref[...], 0.0)

    @pl.when(i == 0)
    def _():
        sums[...] = jnp.zeros_like(sums)
        cnt[...] = jnp.zeros_like(cnt)

    onehot = (batch_ref[...] ==
              lax.broadcasted_iota(jnp.int32, (1, G), 1)).astype(jnp.float32)
    sums[...] += lax.dot_general(onehot, y, (((0,), (0,)), ((), ())),
                                 preferred_element_type=jnp.float32)
    cnt[...] += lax.dot_general(onehot, jnp.ones((RB, 1), jnp.float32),
                                (((0,), (0,)), ((), ())),
                                preferred_element_type=jnp.float32)

    @pl.when(i == NB - 1)
    def _():
        pooled = sums[...] / jnp.maximum(cnt[...], 1.0)
        h = jnp.maximum(jnp.dot(pooled, lw1_ref[...],
                                preferred_element_type=jnp.float32)
                        + lb1_ref[...], 0.0)
        o_ref[...] = jnp.dot(h, lw2_ref[...],
                             preferred_element_type=jnp.float32) + lb2_ref[...]


def _head_call(conv, st, g, be, batchcol, lw1, lb1, lw2, lb2):
    return pl.pallas_call(
        _head_body,
        grid=(NB,),
        in_specs=[
            pl.BlockSpec((RB, H), lambda i: (i, 0)),
            pl.BlockSpec((2, H), lambda i: (0, 0)),
            pl.BlockSpec((1, H), lambda i: (0, 0)),
            pl.BlockSpec((1, H), lambda i: (0, 0)),
            pl.BlockSpec((RB, 1), lambda i: (i, 0)),
            pl.BlockSpec((H, G), lambda i: (0, 0)),
            pl.BlockSpec((1, G), lambda i: (0, 0)),
            pl.BlockSpec((G, 1), lambda i: (0, 0)),
            pl.BlockSpec((1, 1), lambda i: (0, 0)),
        ],
        out_specs=pl.BlockSpec((G, 1), lambda i: (0, 0)),
        out_shape=jax.ShapeDtypeStruct((G, 1), jnp.float32),
        scratch_shapes=[
            pltpu.VMEM((G, H), jnp.float32),
            pltpu.VMEM((G, 1), jnp.float32),
        ],
    )(conv, st, g, be, batchcol, lw1, lb1, lw2, lb2)


def kernel(x, edge_index, batch, W1, b1, W2, b2, W3, b3,
           g1, be1, g2, be2, g3, be3, lw1, lb1, lw2, lb2):
    loop = jnp.arange(N, dtype=jnp.int32)
    pad = TOT - (E + N)
    src_t = jnp.concatenate(
        [edge_index[0], loop, jnp.zeros((pad,), jnp.int32)]).reshape(NS, NCH, CH)
    src_all = jnp.stack([src_t, src_t + N])
    dst_all = jnp.concatenate(
        [edge_index[1], loop, jnp.full((pad,), N, jnp.int32)]).reshape(NS, NCH, CH)
    zcol = jnp.zeros((RPT,), jnp.float32)
    zrows = jnp.zeros((RPT_A, HF), jnp.bfloat16)

    degs = _deg_kernel(dst_all.reshape(NW, NCH // 2, CH),
                       zcol).reshape(NC, N_PAD, 1)

    hp = _mm_call(x, W1, degs)
    acc = _scatter_kernel(hp.reshape(NC * N, HF), src_all, dst_all, zrows)
    conv, st = _post_call(acc, degs, b1.reshape(1, H))

    hp = _bnmm_call(conv, st, g1.reshape(1, H), be1.reshape(1, H), W2, degs)
    acc = _scatter_kernel(hp.reshape(NC * N, HF), src_all, dst_all, zrows)
    conv, st = _post_call(acc, degs, b2.reshape(1, H))

    hp = _bnmm_call(conv, st, g2.reshape(1, H), be2.reshape(1, H), W3, degs)
    acc = _scatter_kernel(hp.reshape(NC * N, HF), src_all, dst_all, zrows)
    conv, st = _post_call(acc, degs, b3.reshape(1, H))

    return _head_call(conv, st, g3.reshape(1, H), be3.reshape(1, H),
                      batch.reshape(N, 1).astype(jnp.int32),
                      lw1, lb1.reshape(1, G), lw2, lb2.reshape(1, 1))
